# baseline (speedup 1.0000x reference)
import functools

import jax
import jax.numpy as jnp
from jax import lax
from jax.experimental import pallas as pl
from jax.experimental.pallas import tpu as pltpu
from jax.experimental.pallas import tpu_sc as plsc

CORE0_SHARE = 0.4608
NC = 2
NS = 16
NW = NC * NS
K = 128
S0 = 624
ZR = 16


def _sc_degree(dst3, n):
    c_chunks = dst3.shape[1]
    mesh = plsc.VectorSubcoreMesh(core_axis_name="c", subcore_axis_name="s")

    @functools.partial(
        pl.kernel,
        out_type=jax.ShapeDtypeStruct((NC, n, 128), jnp.float32),
        mesh=mesh,
        scratch_types=[
            pltpu.VMEM((c_chunks, K), jnp.int32),
            pltpu.VMEM((K, 128), jnp.float32),
            pltpu.VMEM((ZR, 128), jnp.float32),
            pltpu.VMEM_SHARED((n + 8, 128), jnp.float32),
        ],
    )
    def deg_kernel(dst_hbm, out_hbm, idx_v, ones_v, zbuf_v, acc_sh):
        cid = lax.axis_index("c")
        sid = lax.axis_index("s")
        wid = cid * NS + sid
        rem = n - NS * S0

        @pl.loop(0, ZR)
        def _(i):
            @pl.loop(0, 8)
            def _(j):
                zbuf_v[i, pl.ds(j * 16, 16)] = jnp.zeros((16,), jnp.float32)

        @pl.loop(0, K)
        def _(i):
            @pl.loop(0, 8)
            def _(j):
                ones_v[i, pl.ds(j * 16, 16)] = jnp.ones((16,), jnp.float32)

        @pl.loop(0, S0 // ZR)
        def _(t):
            pltpu.sync_copy(zbuf_v, acc_sh.at[pl.ds(sid * S0 + t * ZR, ZR)])

        @pl.when(sid == NS - 1)
        def _():
            pltpu.sync_copy(zbuf_v.at[pl.ds(0, rem)],
                            acc_sh.at[pl.ds(NS * S0, rem)])

        plsc.subcore_barrier()

        pltpu.sync_copy(dst_hbm.at[wid], idx_v)

        @pl.loop(0, c_chunks)
        def _(j):
            pltpu.sync_copy(ones_v, acc_sh.at[idx_v.at[j]], add=True)

        plsc.subcore_barrier()
        pltpu.sync_copy(acc_sh.at[pl.ds(sid * S0, S0)],
                        out_hbm.at[cid, pl.ds(sid * S0, S0)])

        @pl.when(sid == NS - 1)
        def _():
            pltpu.sync_copy(acc_sh.at[pl.ds(NS * S0, rem)],
                            out_hbm.at[cid, pl.ds(NS * S0, rem)])

    return deg_kernel(dst3)


def _sc_aggregate(g, src3, dst3, cnt_a, cnt_b, n):
    c_chunks = src3.shape[1]
    mesh = plsc.VectorSubcoreMesh(core_axis_name="c", subcore_axis_name="s")

    @functools.partial(
        pl.kernel,
        out_type=jax.ShapeDtypeStruct((NC, n, 128), jnp.float32),
        mesh=mesh,
        scratch_types=[
            pltpu.VMEM((c_chunks, K), jnp.int32),
            pltpu.VMEM((c_chunks, K), jnp.int32),
            pltpu.VMEM((K, 128), jnp.float32),
            pltpu.VMEM((ZR, 128), jnp.float32),
            pltpu.VMEM_SHARED((n + 8, 128), jnp.float32),
            pltpu.SemaphoreType.DMA,
        ],
    )
    def agg_kernel(g_hbm, src_hbm, dst_hbm, out_hbm,
                   idx_s, idx_d, rows_v, zbuf_v, acc_sh, sem):
        cid = lax.axis_index("c")
        sid = lax.axis_index("s")
        wid = cid * NS + sid
        rem = n - NS * S0

        @pl.loop(0, ZR)
        def _(i):
            @pl.loop(0, 8)
            def _(j):
                zbuf_v[i, pl.ds(j * 16, 16)] = jnp.zeros((16,), jnp.float32)

        @pl.loop(0, S0 // ZR)
        def _(t):
            pltpu.sync_copy(zbuf_v, acc_sh.at[pl.ds(sid * S0 + t * ZR, ZR)])

        @pl.when(sid == NS - 1)
        def _():
            pltpu.sync_copy(zbuf_v.at[pl.ds(0, rem)],
                            acc_sh.at[pl.ds(NS * S0, rem)])

        plsc.subcore_barrier()

        pltpu.sync_copy(src_hbm.at[wid], idx_s)
        pltpu.sync_copy(dst_hbm.at[wid], idx_d)
        my_count = jnp.where(cid == 0, cnt_a, cnt_b)

        @pl.loop(0, c_chunks)
        def _(j):
            @pl.when(j < my_count)
            def _():
                pltpu.async_copy(g_hbm.at[idx_s.at[j]], rows_v, sem).wait()
                pltpu.sync_copy(rows_v, acc_sh.at[idx_d.at[j]], add=True)

        plsc.subcore_barrier()
        pltpu.sync_copy(acc_sh.at[pl.ds(sid * S0, S0)],
                        out_hbm.at[cid, pl.ds(sid * S0, S0)])

        @pl.when(sid == NS - 1)
        def _():
            pltpu.sync_copy(acc_sh.at[pl.ds(NS * S0, rem)],
                            out_hbm.at[cid, pl.ds(NS * S0, rem)])

    return agg_kernel(g, src3, dst3)


IB = 1000


def _tc_matmul(x, w):
    m, k = x.shape
    h = w.shape[1]

    def body(x_ref, w_ref, o_ref):
        o_ref[...] = jnp.dot(x_ref[...], w_ref[...],
                             preferred_element_type=jnp.float32)

    return pl.pallas_call(
        body,
        grid=(m // IB,),
        in_specs=[pl.BlockSpec((IB, k), lambda i: (i, 0)),
                  pl.BlockSpec((k, h), lambda i: (0, 0))],
        out_specs=pl.BlockSpec((IB, h), lambda i: (i, 0)),
        out_shape=jax.ShapeDtypeStruct((m, h), jnp.float32),
    )(x, w)


def _tc_dinv_g(degp, h1):
    n, f = h1.shape

    def body(d_ref, h_ref, dinv_ref, g_ref):
        deg = d_ref[0, :, 0:1] + d_ref[1, :, 0:1] + 1.0
        dv = lax.rsqrt(deg)
        dinv_ref[...] = dv
        g_ref[...] = h_ref[...] * dv

    return pl.pallas_call(
        body,
        grid=(n // IB,),
        in_specs=[pl.BlockSpec((NC, IB, 128), lambda i: (0, i, 0)),
                  pl.BlockSpec((IB, f), lambda i: (i, 0))],
        out_specs=[pl.BlockSpec((IB, 1), lambda i: (i, 0)),
                   pl.BlockSpec((IB, f), lambda i: (i, 0))],
        out_shape=[jax.ShapeDtypeStruct((n, 1), jnp.float32),
                   jax.ShapeDtypeStruct((n, f), jnp.float32)],
    )(degp, h1)


def _tc_layer(accp, h, dinv, b2d, w_next):
    n, f = h.shape
    h_out = w_next.shape[1]

    def body(a_ref, h_ref, d_ref, b_ref, w_ref, hn_ref, gn_ref):
        dv = d_ref[...]
        z = dv * (a_ref[0] + a_ref[1]) + dv * dv * h_ref[...] + b_ref[...]
        r = jnp.maximum(z, 0.0)
        hn = jnp.dot(r, w_ref[...], preferred_element_type=jnp.float32)
        hn_ref[...] = hn
        gn_ref[...] = hn * dv

    return pl.pallas_call(
        body,
        grid=(n // IB,),
        in_specs=[pl.BlockSpec((NC, IB, f), lambda i: (0, i, 0)),
                  pl.BlockSpec((IB, f), lambda i: (i, 0)),
                  pl.BlockSpec((IB, 1), lambda i: (i, 0)),
                  pl.BlockSpec((1, f), lambda i: (0, 0)),
                  pl.BlockSpec((f, h_out), lambda i: (0, 0))],
        out_specs=[pl.BlockSpec((IB, h_out), lambda i: (i, 0)),
                   pl.BlockSpec((IB, h_out), lambda i: (i, 0))],
        out_shape=[jax.ShapeDtypeStruct((n, h_out), jnp.float32),
                   jax.ShapeDtypeStruct((n, h_out), jnp.float32)],
    )(accp, h, dinv, b2d, w_next)


def _tc_final(accp, h, dinv, b2d, batch3d, wdc, bdc2d, num_graphs):
    n, f = h.shape
    out_dim = wdc.shape[1]
    steps = n // IB

    def body(a_ref, h_ref, d_ref, b_ref, bat_ref, wdc_ref, bdc_ref,
             o_ref, sums_s, cnts_s):
        i = pl.program_id(0)

        @pl.when(i == 0)
        def _():
            sums_s[...] = jnp.zeros_like(sums_s)
            cnts_s[...] = jnp.zeros_like(cnts_s)

        dv = d_ref[...]
        z = dv * (a_ref[0] + a_ref[1]) + dv * dv * h_ref[...] + b_ref[...]
        seg = lax.broadcasted_iota(jnp.int32, (num_graphs, IB), 0)
        m_t = jnp.where(seg == jnp.broadcast_to(bat_ref[0], (num_graphs, IB)),
                        1.0, 0.0).astype(jnp.float32)
        sums_s[...] += jnp.dot(m_t, z, preferred_element_type=jnp.float32)
        cnts_s[...] += jnp.sum(m_t, axis=1, keepdims=True)

        @pl.when(i == steps - 1)
        def _():
            pooled = sums_s[...] / jnp.maximum(cnts_s[...], 1.0)
            logits = jnp.dot(pooled, wdc_ref[...],
                             preferred_element_type=jnp.float32) + bdc_ref[...]
            mx = jnp.max(logits, axis=-1, keepdims=True)
            e = jnp.exp(logits - mx)
            p = e / jnp.sum(e, axis=-1, keepdims=True)
            o_ref[...] = jnp.where(p >= 0.5, 1.0, 0.0).astype(jnp.float32)

    return pl.pallas_call(
        body,
        grid=(steps,),
        in_specs=[pl.BlockSpec((NC, IB, f), lambda i: (0, i, 0)),
                  pl.BlockSpec((IB, f), lambda i: (i, 0)),
                  pl.BlockSpec((IB, 1), lambda i: (i, 0)),
                  pl.BlockSpec((1, f), lambda i: (0, 0)),
                  pl.BlockSpec((1, 1, IB), lambda i: (i, 0, 0)),
                  pl.BlockSpec((f, out_dim), lambda i: (0, 0)),
                  pl.BlockSpec((1, out_dim), lambda i: (0, 0))],
        out_specs=pl.BlockSpec((num_graphs, out_dim), lambda i: (0, 0)),
        out_shape=jax.ShapeDtypeStruct((num_graphs, out_dim), jnp.float32),
        scratch_shapes=[pltpu.VMEM((num_graphs, f), jnp.float32),
                        pltpu.VMEM((num_graphs, 1), jnp.float32)],
    )(accp, h, dinv, b2d, batch3d, wdc, bdc2d)


def kernel(x, edge_index, batch, W1, b1, W2, b2, W3, b3, Wdc, bdc):
    n = x.shape[0]
    e = edge_index.shape[1]
    num_graphs = 64

    src = edge_index[0]
    dst = edge_index[1]
    t_chunks = -(-e // K)
    a = int(round(t_chunks * CORE0_SHARE / NS))
    b = -(-max(t_chunks - NS * a, 0) // NS)
    cmax = max(a, b, 1)
    e_pad = NS * (a + b) * K
    pad = e_pad - e
    src_p = jnp.concatenate([src, jnp.zeros((pad,), jnp.int32)])
    dst_p = jnp.concatenate([dst, jnp.full((pad,), n, jnp.int32)])

    def worker_layout(flat, fill):
        pa = flat[:NS * a * K].reshape(NS, a, K)
        pb = flat[NS * a * K:].reshape(NS, b, K)
        fa = jnp.full((NS, cmax - a, K), fill, jnp.int32)
        fb = jnp.full((NS, cmax - b, K), fill, jnp.int32)
        return jnp.concatenate(
            [jnp.concatenate([pa, fa], axis=1),
             jnp.concatenate([pb, fb], axis=1)], axis=0)

    src3 = worker_layout(src_p, 0)
    dst3 = worker_layout(dst_p, n)

    c_deg = -(-e // (NW * K))
    pad_d = NW * K * c_deg - e
    dst3d = jnp.concatenate([dst, jnp.full((pad_d,), n, jnp.int32)])
    dst3d = dst3d.reshape(NW, c_deg, K)

    b1_2d = b1.reshape(1, -1)
    b2_2d = b2.reshape(1, -1)
    b3_2d = b3.reshape(1, -1)
    bdc2d = bdc.reshape(1, -1)
    batch3d = batch.reshape(n // IB, 1, IB)

    degp = _sc_degree(dst3d, n)
    h1 = _tc_matmul(x, W1)
    dinv, g1 = _tc_dinv_g(degp, h1)

    acc1 = _sc_aggregate(g1, src3, dst3, a, b, n)
    h2, g2 = _tc_layer(acc1, h1, dinv, b1_2d, W2)

    acc2 = _sc_aggregate(g2, src3, dst3, a, b, n)
    h3, g3 = _tc_layer(acc2, h2, dinv, b2_2d, W3)

    acc3 = _sc_aggregate(g3, src3, dst3, a, b, n)
    return _tc_final(acc3, h3, dinv, b3_2d, batch3d, Wdc, bdc2d, num_graphs)


# speedup vs baseline: 1.0375x; 1.0375x over previous
import functools

import jax
import jax.numpy as jnp
from jax import lax
from jax.experimental import pallas as pl
from jax.experimental.pallas import tpu as pltpu
from jax.experimental.pallas import tpu_sc as plsc

CORE0_SHARE = 0.4096
NC = 2
NS = 16
NW = NC * NS
K = 128
S0 = 624
ZR = 16


def _sc_degree(dst3, n):
    c_chunks = dst3.shape[1]
    mesh = plsc.VectorSubcoreMesh(core_axis_name="c", subcore_axis_name="s")

    @functools.partial(
        pl.kernel,
        out_type=jax.ShapeDtypeStruct((NC, n, 128), jnp.float32),
        mesh=mesh,
        scratch_types=[
            pltpu.VMEM((c_chunks, K), jnp.int32),
            pltpu.VMEM((K, 128), jnp.float32),
            pltpu.VMEM((ZR, 128), jnp.float32),
            pltpu.VMEM_SHARED((n + 8, 128), jnp.float32),
        ],
    )
    def deg_kernel(dst_hbm, out_hbm, idx_v, ones_v, zbuf_v, acc_sh):
        cid = lax.axis_index("c")
        sid = lax.axis_index("s")
        wid = cid * NS + sid
        rem = n - NS * S0

        @pl.loop(0, ZR)
        def _(i):
            @pl.loop(0, 8)
            def _(j):
                zbuf_v[i, pl.ds(j * 16, 16)] = jnp.zeros((16,), jnp.float32)

        @pl.loop(0, K)
        def _(i):
            @pl.loop(0, 8)
            def _(j):
                ones_v[i, pl.ds(j * 16, 16)] = jnp.ones((16,), jnp.float32)

        @pl.loop(0, S0 // ZR)
        def _(t):
            pltpu.sync_copy(zbuf_v, acc_sh.at[pl.ds(sid * S0 + t * ZR, ZR)])

        @pl.when(sid == NS - 1)
        def _():
            pltpu.sync_copy(zbuf_v.at[pl.ds(0, rem)],
                            acc_sh.at[pl.ds(NS * S0, rem)])

        plsc.subcore_barrier()

        pltpu.sync_copy(dst_hbm.at[wid], idx_v)

        @pl.loop(0, c_chunks)
        def _(j):
            pltpu.sync_copy(ones_v, acc_sh.at[idx_v.at[j]], add=True)

        plsc.subcore_barrier()
        pltpu.sync_copy(acc_sh.at[pl.ds(sid * S0, S0)],
                        out_hbm.at[cid, pl.ds(sid * S0, S0)])

        @pl.when(sid == NS - 1)
        def _():
            pltpu.sync_copy(acc_sh.at[pl.ds(NS * S0, rem)],
                            out_hbm.at[cid, pl.ds(NS * S0, rem)])

    return deg_kernel(dst3)


def _sc_aggregate(g, src3, dst3, cnt_a, cnt_b, n):
    c_chunks = src3.shape[1]
    mesh = plsc.VectorSubcoreMesh(core_axis_name="c", subcore_axis_name="s")

    @functools.partial(
        pl.kernel,
        out_type=jax.ShapeDtypeStruct((NC, n, 128), jnp.float32),
        mesh=mesh,
        scratch_types=[
            pltpu.VMEM((c_chunks, K), jnp.int32),
            pltpu.VMEM((c_chunks, K), jnp.int32),
            pltpu.VMEM((K, 128), jnp.float32),
            pltpu.VMEM((ZR, 128), jnp.float32),
            pltpu.VMEM_SHARED((n + 8, 128), jnp.float32),
            pltpu.SemaphoreType.DMA,
        ],
    )
    def agg_kernel(g_hbm, src_hbm, dst_hbm, out_hbm,
                   idx_s, idx_d, rows_v, zbuf_v, acc_sh, sem):
        cid = lax.axis_index("c")
        sid = lax.axis_index("s")
        wid = cid * NS + sid
        rem = n - NS * S0

        @pl.loop(0, ZR)
        def _(i):
            @pl.loop(0, 8)
            def _(j):
                zbuf_v[i, pl.ds(j * 16, 16)] = jnp.zeros((16,), jnp.float32)

        @pl.loop(0, S0 // ZR)
        def _(t):
            pltpu.sync_copy(zbuf_v, acc_sh.at[pl.ds(sid * S0 + t * ZR, ZR)])

        @pl.when(sid == NS - 1)
        def _():
            pltpu.sync_copy(zbuf_v.at[pl.ds(0, rem)],
                            acc_sh.at[pl.ds(NS * S0, rem)])

        plsc.subcore_barrier()

        pltpu.sync_copy(src_hbm.at[wid], idx_s)
        pltpu.sync_copy(dst_hbm.at[wid], idx_d)
        my_count = jnp.where(cid == 0, cnt_a, cnt_b)

        @pl.loop(0, c_chunks)
        def _(j):
            @pl.when(j < my_count)
            def _():
                pltpu.async_copy(g_hbm.at[idx_s.at[j]], rows_v, sem).wait()
                pltpu.sync_copy(rows_v, acc_sh.at[idx_d.at[j]], add=True)

        plsc.subcore_barrier()
        pltpu.sync_copy(acc_sh.at[pl.ds(sid * S0, S0)],
                        out_hbm.at[cid, pl.ds(sid * S0, S0)])

        @pl.when(sid == NS - 1)
        def _():
            pltpu.sync_copy(acc_sh.at[pl.ds(NS * S0, rem)],
                            out_hbm.at[cid, pl.ds(NS * S0, rem)])

    return agg_kernel(g, src3, dst3)


IB = 1000


def _tc_matmul(x, w):
    m, k = x.shape
    h = w.shape[1]

    def body(x_ref, w_ref, o_ref):
        o_ref[...] = jnp.dot(x_ref[...], w_ref[...],
                             preferred_element_type=jnp.float32)

    return pl.pallas_call(
        body,
        grid=(m // IB,),
        in_specs=[pl.BlockSpec((IB, k), lambda i: (i, 0)),
                  pl.BlockSpec((k, h), lambda i: (0, 0))],
        out_specs=pl.BlockSpec((IB, h), lambda i: (i, 0)),
        out_shape=jax.ShapeDtypeStruct((m, h), jnp.float32),
    )(x, w)


def _tc_dinv_g(degp, h1):
    n, f = h1.shape

    def body(d_ref, h_ref, dinv_ref, g_ref):
        deg = d_ref[0, :, 0:1] + d_ref[1, :, 0:1] + 1.0
        dv = lax.rsqrt(deg)
        dinv_ref[...] = dv
        g_ref[...] = h_ref[...] * dv

    return pl.pallas_call(
        body,
        grid=(n // IB,),
        in_specs=[pl.BlockSpec((NC, IB, 128), lambda i: (0, i, 0)),
                  pl.BlockSpec((IB, f), lambda i: (i, 0))],
        out_specs=[pl.BlockSpec((IB, 1), lambda i: (i, 0)),
                   pl.BlockSpec((IB, f), lambda i: (i, 0))],
        out_shape=[jax.ShapeDtypeStruct((n, 1), jnp.float32),
                   jax.ShapeDtypeStruct((n, f), jnp.float32)],
    )(degp, h1)


def _tc_layer(accp, h, dinv, b2d, w_next):
    n, f = h.shape
    h_out = w_next.shape[1]

    def body(a_ref, h_ref, d_ref, b_ref, w_ref, hn_ref, gn_ref):
        dv = d_ref[...]
        z = dv * (a_ref[0] + a_ref[1]) + dv * dv * h_ref[...] + b_ref[...]
        r = jnp.maximum(z, 0.0)
        hn = jnp.dot(r, w_ref[...], preferred_element_type=jnp.float32)
        hn_ref[...] = hn
        gn_ref[...] = hn * dv

    return pl.pallas_call(
        body,
        grid=(n // IB,),
        in_specs=[pl.BlockSpec((NC, IB, f), lambda i: (0, i, 0)),
                  pl.BlockSpec((IB, f), lambda i: (i, 0)),
                  pl.BlockSpec((IB, 1), lambda i: (i, 0)),
                  pl.BlockSpec((1, f), lambda i: (0, 0)),
                  pl.BlockSpec((f, h_out), lambda i: (0, 0))],
        out_specs=[pl.BlockSpec((IB, h_out), lambda i: (i, 0)),
                   pl.BlockSpec((IB, h_out), lambda i: (i, 0))],
        out_shape=[jax.ShapeDtypeStruct((n, h_out), jnp.float32),
                   jax.ShapeDtypeStruct((n, h_out), jnp.float32)],
    )(accp, h, dinv, b2d, w_next)


def _tc_final(accp, h, dinv, b2d, batch3d, wdc, bdc2d, num_graphs):
    n, f = h.shape
    out_dim = wdc.shape[1]
    steps = n // IB

    def body(a_ref, h_ref, d_ref, b_ref, bat_ref, wdc_ref, bdc_ref,
             o_ref, sums_s, cnts_s):
        i = pl.program_id(0)

        @pl.when(i == 0)
        def _():
            sums_s[...] = jnp.zeros_like(sums_s)
            cnts_s[...] = jnp.zeros_like(cnts_s)

        dv = d_ref[...]
        z = dv * (a_ref[0] + a_ref[1]) + dv * dv * h_ref[...] + b_ref[...]
        seg = lax.broadcasted_iota(jnp.int32, (num_graphs, IB), 0)
        m_t = jnp.where(seg == jnp.broadcast_to(bat_ref[0], (num_graphs, IB)),
                        1.0, 0.0).astype(jnp.float32)
        sums_s[...] += jnp.dot(m_t, z, preferred_element_type=jnp.float32)
        cnts_s[...] += jnp.sum(m_t, axis=1, keepdims=True)

        @pl.when(i == steps - 1)
        def _():
            pooled = sums_s[...] / jnp.maximum(cnts_s[...], 1.0)
            logits = jnp.dot(pooled, wdc_ref[...],
                             preferred_element_type=jnp.float32) + bdc_ref[...]
            mx = jnp.max(logits, axis=-1, keepdims=True)
            e = jnp.exp(logits - mx)
            p = e / jnp.sum(e, axis=-1, keepdims=True)
            o_ref[...] = jnp.where(p >= 0.5, 1.0, 0.0).astype(jnp.float32)

    return pl.pallas_call(
        body,
        grid=(steps,),
        in_specs=[pl.BlockSpec((NC, IB, f), lambda i: (0, i, 0)),
                  pl.BlockSpec((IB, f), lambda i: (i, 0)),
                  pl.BlockSpec((IB, 1), lambda i: (i, 0)),
                  pl.BlockSpec((1, f), lambda i: (0, 0)),
                  pl.BlockSpec((1, 1, IB), lambda i: (i, 0, 0)),
                  pl.BlockSpec((f, out_dim), lambda i: (0, 0)),
                  pl.BlockSpec((1, out_dim), lambda i: (0, 0))],
        out_specs=pl.BlockSpec((num_graphs, out_dim), lambda i: (0, 0)),
        out_shape=jax.ShapeDtypeStruct((num_graphs, out_dim), jnp.float32),
        scratch_shapes=[pltpu.VMEM((num_graphs, f), jnp.float32),
                        pltpu.VMEM((num_graphs, 1), jnp.float32)],
    )(accp, h, dinv, b2d, batch3d, wdc, bdc2d)


def kernel(x, edge_index, batch, W1, b1, W2, b2, W3, b3, Wdc, bdc):
    n = x.shape[0]
    e = edge_index.shape[1]
    num_graphs = 64

    src = edge_index[0]
    dst = edge_index[1]
    t_chunks = -(-e // K)
    a = int(round(t_chunks * CORE0_SHARE / NS))
    b = -(-max(t_chunks - NS * a, 0) // NS)
    cmax = max(a, b, 1)
    e_pad = NS * (a + b) * K
    pad = e_pad - e
    src_p = jnp.concatenate([src, jnp.zeros((pad,), jnp.int32)])
    dst_p = jnp.concatenate([dst, jnp.full((pad,), n, jnp.int32)])

    def worker_layout(flat, fill):
        pa = flat[:NS * a * K].reshape(NS, a, K)
        pb = flat[NS * a * K:].reshape(NS, b, K)
        fa = jnp.full((NS, cmax - a, K), fill, jnp.int32)
        fb = jnp.full((NS, cmax - b, K), fill, jnp.int32)
        return jnp.concatenate(
            [jnp.concatenate([pa, fa], axis=1),
             jnp.concatenate([pb, fb], axis=1)], axis=0)

    src3 = worker_layout(src_p, 0)
    dst3 = worker_layout(dst_p, n)

    c_deg = -(-e // (NW * K))
    pad_d = NW * K * c_deg - e
    dst3d = jnp.concatenate([dst, jnp.full((pad_d,), n, jnp.int32)])
    dst3d = dst3d.reshape(NW, c_deg, K)

    b1_2d = b1.reshape(1, -1)
    b2_2d = b2.reshape(1, -1)
    b3_2d = b3.reshape(1, -1)
    bdc2d = bdc.reshape(1, -1)
    batch3d = batch.reshape(n // IB, 1, IB)

    degp = _sc_degree(dst3d, n)
    h1 = _tc_matmul(x, W1)
    dinv, g1 = _tc_dinv_g(degp, h1)

    acc1 = _sc_aggregate(g1, src3, dst3, a, b, n)
    h2, g2 = _tc_layer(acc1, h1, dinv, b1_2d, W2)

    acc2 = _sc_aggregate(g2, src3, dst3, a, b, n)
    h3, g3 = _tc_layer(acc2, h2, dinv, b2_2d, W3)

    acc3 = _sc_aggregate(g3, src3, dst3, a, b, n)
    return _tc_final(acc3, h3, dinv, b3_2d, batch3d, Wdc, bdc2d, num_graphs)
